# baseline (speedup 1.0000x reference)
import functools

import jax
import jax.numpy as jnp
from jax.experimental import pallas as pl

_ROWS = 10000
_FEAT = 128
_BLOCK_R = 1000


def _linear_kernel(x_ref, w_ref, b_ref, o_ref):
    xw = jax.lax.dot_general(
        x_ref[...], w_ref[...],
        dimension_numbers=(((1,), (1,)), ((), ())),
        preferred_element_type=jnp.float32,
    )
    o_ref[...] = xw + b_ref[...]


def kernel(edge_index, x, W, b):
    del edge_index
    b2 = b.reshape(1, _FEAT)
    grid = (_ROWS // _BLOCK_R,)
    out = pl.pallas_call(
        _linear_kernel,
        grid=grid,
        in_specs=[
            pl.BlockSpec((_BLOCK_R, _FEAT), lambda i: (i, 0)),
            pl.BlockSpec((_FEAT, _FEAT), lambda i: (0, 0)),
            pl.BlockSpec((1, _FEAT), lambda i: (0, 0)),
        ],
        out_specs=pl.BlockSpec((_BLOCK_R, _FEAT), lambda i: (i, 0)),
        out_shape=jax.ShapeDtypeStruct((_ROWS, _FEAT), jnp.float32),
    )(x, W, b2)
    return out


# speedup vs baseline: 1.3188x; 1.3188x over previous
import functools

import jax
import jax.numpy as jnp
from jax.experimental import pallas as pl

_ROWS = 10000
_FEAT = 128
_BLOCK_R = 2000


def _linear_kernel(x_ref, w_ref, b_ref, o_ref):
    xw = jax.lax.dot_general(
        x_ref[...], w_ref[...],
        dimension_numbers=(((1,), (1,)), ((), ())),
        preferred_element_type=jnp.float32,
    )
    o_ref[...] = xw + b_ref[...]


def kernel(edge_index, x, W, b):
    del edge_index
    b2 = b.reshape(1, _FEAT)
    grid = (_ROWS // _BLOCK_R,)
    out = pl.pallas_call(
        _linear_kernel,
        grid=grid,
        in_specs=[
            pl.BlockSpec((_BLOCK_R, _FEAT), lambda i: (i, 0)),
            pl.BlockSpec((_FEAT, _FEAT), lambda i: (0, 0)),
            pl.BlockSpec((1, _FEAT), lambda i: (0, 0)),
        ],
        out_specs=pl.BlockSpec((_BLOCK_R, _FEAT), lambda i: (i, 0)),
        out_shape=jax.ShapeDtypeStruct((_ROWS, _FEAT), jnp.float32),
    )(x, W, b2)
    return out


# speedup vs baseline: 1.8911x; 1.4339x over previous
import functools

import jax
import jax.numpy as jnp
from jax.experimental import pallas as pl

_ROWS = 10000
_FEAT = 128
_BLOCK_R = 5000


def _linear_kernel(x_ref, w_ref, b_ref, o_ref):
    xw = jax.lax.dot_general(
        x_ref[...], w_ref[...],
        dimension_numbers=(((1,), (1,)), ((), ())),
        preferred_element_type=jnp.float32,
    )
    o_ref[...] = xw + b_ref[...]


def kernel(edge_index, x, W, b):
    del edge_index
    b2 = b.reshape(1, _FEAT)
    grid = (_ROWS // _BLOCK_R,)
    out = pl.pallas_call(
        _linear_kernel,
        grid=grid,
        in_specs=[
            pl.BlockSpec((_BLOCK_R, _FEAT), lambda i: (i, 0)),
            pl.BlockSpec((_FEAT, _FEAT), lambda i: (0, 0)),
            pl.BlockSpec((1, _FEAT), lambda i: (0, 0)),
        ],
        out_specs=pl.BlockSpec((_BLOCK_R, _FEAT), lambda i: (i, 0)),
        out_shape=jax.ShapeDtypeStruct((_ROWS, _FEAT), jnp.float32),
    )(x, W, b2)
    return out
